# baseline (speedup 1.0000x reference)
import jax
import jax.numpy as jnp
from jax import lax
from jax.experimental import pallas as pl
from jax.experimental.pallas import tpu as pltpu
from jax.experimental.pallas import tpu_sc as plsc

DIM = 768
NUM_CORES = 2
NUM_SUBCORES = 16
NUM_WORKERS = NUM_CORES * NUM_SUBCORES
NBUF = 4
CHUNK = 40


def kernel(channel_seq, table):
    nseq, seqlen = channel_seq.shape
    num_indices = channel_seq.size
    flat_idx = channel_seq.T.reshape(num_indices).astype(jnp.int32)
    b_per_w = num_indices // NUM_WORKERS
    nchunks = b_per_w // CHUNK
    mesh = plsc.VectorSubcoreMesh(core_axis_name="c", subcore_axis_name="s")

    @jax.jit
    def gather(table, idx):
        @pl.kernel(
            out_type=jax.ShapeDtypeStruct((num_indices, DIM), table.dtype),
            mesh=mesh,
            scratch_types=(
                [pltpu.VMEM((b_per_w,), jnp.int32)]
                + [pltpu.VMEM((CHUNK, DIM), jnp.float32) for _ in range(NBUF)]
                + [pltpu.SemaphoreType.DMA for _ in range(2 * NBUF)]
            ),
        )
        def sc_gather(table_hbm, idx_hbm, out_hbm, idx_v, *bufs_and_sems):
            rows = bufs_and_sems[:NBUF]
            gsem = bufs_and_sems[NBUF:2 * NBUF]
            ssem = bufs_and_sems[2 * NBUF:]
            wid = lax.axis_index("s") * NUM_CORES + lax.axis_index("c")
            base = wid * b_per_w
            pltpu.sync_copy(idx_hbm.at[pl.ds(base, b_per_w)], idx_v)

            def start_gather(g, b):
                pltpu.async_copy(
                    table_hbm.at[idx_v.at[pl.ds(g * CHUNK, CHUNK)]], rows[b], gsem[b])

            def start_store(g, b):
                pltpu.async_copy(
                    rows[b], out_hbm.at[pl.ds(base + g * CHUNK, CHUNK)], ssem[b])

            def wait_gather(b):
                pltpu.make_async_copy(table_hbm.at[idx_v.at[pl.ds(0, CHUNK)]],
                                      rows[b], gsem[b]).wait()

            def wait_store(b):
                pltpu.make_async_copy(rows[b], out_hbm.at[pl.ds(base, CHUNK)],
                                      ssem[b]).wait()

            K = 3

            for b in range(K):
                start_gather(b, b)

            @pl.loop(0, nchunks, step=NBUF)
            def _(g0):
                for b in range(NBUF):
                    g = g0 + b
                    bk = (b + K) % NBUF

                    @pl.when(jnp.logical_and(g + K < nchunks, g + K >= NBUF))
                    def _():
                        wait_store(bk)

                    @pl.when(g + K < nchunks)
                    def _():
                        start_gather(g + K, bk)

                    wait_gather(b)
                    start_store(g, b)

            for b in range(NBUF):
                wait_store(b)

        return sc_gather(table, idx)

    out = gather(table, flat_idx)
    return out.reshape(seqlen, nseq, DIM).transpose(1, 0, 2)
